# baseline (speedup 1.0000x reference)
import functools
import jax
import jax.numpy as jnp
from jax import lax
from jax.experimental import pallas as pl
from jax.experimental.pallas import tpu as pltpu
from jax.experimental.pallas import tpu_sc as plsc

_T = 512
_H = 1024
_I = 2048
_E = 8
_K = 2
_B = 256
_NB = 12
_PMAX = _NB * _B
_IB = 1024
_NI = _I // _IB
_NP = _K * _T

_NC = 2
_NS = 16
_NW = _NC * _NS
_PPW = _NP // _NW
_TPW = _T // _NW


def _route_kernel(logits_ref, meta_ref, dest_ref, wall_ref):
    logits = logits_ref[...]
    m1 = jnp.max(logits, axis=1, keepdims=True)
    eiota = lax.broadcasted_iota(jnp.int32, (_T, _E), 1)
    nE = jnp.int32(_E)
    i1 = jnp.min(jnp.where(logits == m1, eiota, nE), axis=1, keepdims=True)
    masked = jnp.where(eiota == i1, -jnp.inf, logits)
    m2 = jnp.max(masked, axis=1, keepdims=True)
    i2 = jnp.min(jnp.where(masked == m2, eiota, nE), axis=1, keepdims=True)
    p2 = jnp.exp(m2 - m1)
    denom = 1.0 + p2
    w_first = 1.0 / denom
    w_second = p2 / denom

    e_all = jnp.concatenate([i1, i2], axis=0)
    w_all = jnp.concatenate([w_first, w_second], axis=0)

    piota = lax.broadcasted_iota(jnp.int32, (_NP, _E), 1)
    M = (e_all == piota).astype(jnp.float32)

    r0 = lax.broadcasted_iota(jnp.int32, (_NP, _NP), 0)
    r1 = lax.broadcasted_iota(jnp.int32, (_NP, _NP), 1)
    L = (r1 < r0).astype(jnp.float32)
    dn = (((1,), (0,)), ((), ()))
    exclcum = lax.dot_general(L, M, dn, preferred_element_type=jnp.float32)

    cnt = jnp.sum(M, axis=0, keepdims=True)
    padded = jnp.ceil(cnt * (1.0 / _B)) * _B
    u0 = lax.broadcasted_iota(jnp.int32, (_E, _E), 0)
    u1 = lax.broadcasted_iota(jnp.int32, (_E, _E), 1)
    U = (u0 < u1).astype(jnp.float32)
    offs = lax.dot_general(padded, U, dn, preferred_element_type=jnp.float32)

    offs_p = jnp.sum(M * offs, axis=1, keepdims=True)
    rank_p = jnp.sum(M * exclcum, axis=1, keepdims=True)
    dest = offs_p + rank_p

    cum = offs + padded
    total = jnp.sum(padded, axis=1, keepdims=True)
    la = jnp.sum((cum < total).astype(jnp.float32), axis=1, keepdims=True)

    bB = lax.broadcasted_iota(jnp.int32, (1, 32), 1).astype(jnp.float32) * _B
    be = jnp.zeros((1, 32), jnp.float32)
    for e in range(_E):
        ce = lax.slice(cum, (0, e), (1, e + 1))
        be = be + (bB >= ce).astype(jnp.float32)
    be = jnp.where(bB < total, be, la)
    nab = total * (1.0 / _B)
    vend = offs + cnt
    vend32 = jnp.zeros((1, 32), jnp.float32)
    lane = lax.broadcasted_iota(jnp.int32, (1, 32), 1)
    for e in range(_E):
        ve = lax.slice(vend, (0, e), (1, e + 1))
        vend32 = vend32 + jnp.where(lane == 16 + e, ve, 0.0)
    meta = jnp.where(lane < _NB, be, jnp.where(lane < 16, nab, vend32))
    meta_ref[...] = meta.astype(jnp.int32)
    dest_ref[...] = dest.astype(jnp.int32)
    wall_ref[...] = w_all


def _route(router_logits):
    return pl.pallas_call(
        _route_kernel,
        in_specs=[pl.BlockSpec((_T, _E), lambda: (0, 0))],
        out_specs=[
            pl.BlockSpec((1, 32), lambda: (0, 0)),
            pl.BlockSpec((_NP, 1), lambda: (0, 0)),
            pl.BlockSpec((_NP, 1), lambda: (0, 0)),
        ],
        out_shape=[
            jax.ShapeDtypeStruct((1, 32), jnp.int32),
            jax.ShapeDtypeStruct((_NP, 1), jnp.int32),
            jax.ShapeDtypeStruct((_NP, 1), jnp.float32),
        ],
    )(router_logits)


@functools.cache
def _make_sc_scatter():
    @functools.partial(
        pl.kernel,
        mesh=plsc.VectorSubcoreMesh(core_axis_name="c", subcore_axis_name="s"),
        out_type=jax.ShapeDtypeStruct((_PMAX, _H), jnp.float32),
        scratch_types=[
            pltpu.VMEM((_PPW,), jnp.int32),
            pltpu.VMEM((_PPW, _H), jnp.float32),
            pltpu.SemaphoreType.DMA,
        ],
    )
    def _sc_scatter(x_hbm, dest_hbm, xs_hbm, idx_v, rows_v, sem):
        wid = lax.axis_index("s") * _NC + lax.axis_index("c")
        base = wid * _PPW
        tok_base = lax.rem(base, _T)
        pltpu.sync_copy(dest_hbm.at[pl.ds(base, _PPW)], idx_v)
        pltpu.sync_copy(x_hbm.at[pl.ds(tok_base, _PPW)], rows_v)
        pltpu.async_copy(rows_v, xs_hbm.at[idx_v], sem).wait()

    return _sc_scatter


def _mlp_kernel(meta_ref, xs_ref, dest_ref, wall_ref,
                w1_ref, w3_ref, w2_ref, out_ref, acc):
    b = pl.program_id(0)
    i = pl.program_id(1)
    nab = meta_ref[0, 12]

    @pl.when((b == 0) & (i == 0))
    def _():
        out_ref[...] = jnp.zeros_like(out_ref)

    @pl.when(b < nab)
    def _():
        xs = xs_ref[...]
        dn = (((1,), (1,)), ((), ()))
        g = lax.dot_general(xs, w1_ref[0], dn,
                            preferred_element_type=jnp.float32)
        u = lax.dot_general(xs, w3_ref[0], dn,
                            preferred_element_type=jnp.float32)
        prod = (g * jax.nn.sigmoid(g)) * u
        acc[:, pl.ds(i * _IB, _IB)] = prod

        @pl.when(i == _NI - 1)
        def _():
            yfull = lax.dot_general(acc[...], w2_ref[0], dn,
                                    preferred_element_type=jnp.float32)
            vend = meta_ref[0, 16 + meta_ref[0, b]]
            riota = (lax.broadcasted_iota(jnp.int32, (_B, 1), 0) + b * _B)
            yfull = jnp.where(riota < vend, yfull, 0.0)
            d1 = dest_ref[:_T, :]
            d2 = dest_ref[_T:, :]
            wa1 = wall_ref[:_T, :]
            wa2 = wall_ref[_T:, :]
            q = lax.broadcasted_iota(jnp.int32, (_T, _B), 1) + b * _B
            C = ((q == d1).astype(jnp.float32) * wa1
                 + (q == d2).astype(jnp.float32) * wa2)
            dnc = (((1,), (0,)), ((), ()))
            out_ref[...] += lax.dot_general(
                C, yfull, dnc, preferred_element_type=jnp.float32)


def _mlp(meta, xs_all, dest, wall, W1, W3, W2):
    def w13_map(b, i, m):
        e = m[0, b]
        ii = jnp.where(b < m[0, 12], i, _NI - 1)
        return (e, ii, 0)

    def w2_map(b, i, m):
        return (m[0, b], 0, 0)

    grid_spec = pltpu.PrefetchScalarGridSpec(
        num_scalar_prefetch=1,
        grid=(_NB, _NI),
        in_specs=[
            pl.BlockSpec((_B, _H), lambda b, i, m: (b, 0)),
            pl.BlockSpec((_NP, 1), lambda b, i, m: (0, 0)),
            pl.BlockSpec((_NP, 1), lambda b, i, m: (0, 0)),
            pl.BlockSpec((1, _IB, _H), w13_map),
            pl.BlockSpec((1, _IB, _H), w13_map),
            pl.BlockSpec((1, _H, _I), w2_map),
        ],
        out_specs=pl.BlockSpec((_T, _H), lambda b, i, m: (0, 0)),
        scratch_shapes=[pltpu.VMEM((_B, _I), jnp.float32)],
    )
    return pl.pallas_call(
        _mlp_kernel,
        grid_spec=grid_spec,
        out_shape=jax.ShapeDtypeStruct((_T, _H), jnp.float32),
        compiler_params=pltpu.CompilerParams(
            dimension_semantics=("arbitrary", "arbitrary"),
        ),
    )(meta, xs_all, dest, wall, W1, W3, W2)


def kernel(hidden_states, router_logits, W1, W3, W2):
    x = hidden_states.reshape(-1, _H)
    meta, dest, wall = _route(router_logits)
    dest1 = dest.reshape(_NP)
    xs_all = _make_sc_scatter()(x, dest1)
    return _mlp(meta, xs_all, dest, wall, W1, W3, W2)


# speedup vs baseline: 1.0666x; 1.0666x over previous
import functools
import jax
import jax.numpy as jnp
from jax import lax
from jax.experimental import pallas as pl
from jax.experimental.pallas import tpu as pltpu
from jax.experimental.pallas import tpu_sc as plsc

_T = 512
_H = 1024
_I = 2048
_E = 8
_K = 2
_B = 256
_NB = 12
_PMAX = _NB * _B
_IB = 1024
_NI = _I // _IB
_NP = _K * _T

_NC = 2
_NS = 16
_NW = _NC * _NS
_PPW = _NP // _NW
_TPW = _T // _NW


def _route_kernel(logits_ref, meta_ref, dest_ref, wall_ref):
    logits = logits_ref[...]
    m1 = jnp.max(logits, axis=1, keepdims=True)
    eiota = lax.broadcasted_iota(jnp.int32, (_T, _E), 1)
    nE = jnp.int32(_E)
    i1 = jnp.min(jnp.where(logits == m1, eiota, nE), axis=1, keepdims=True)
    masked = jnp.where(eiota == i1, -jnp.inf, logits)
    m2 = jnp.max(masked, axis=1, keepdims=True)
    i2 = jnp.min(jnp.where(masked == m2, eiota, nE), axis=1, keepdims=True)
    p2 = jnp.exp(m2 - m1)
    denom = 1.0 + p2
    w_first = 1.0 / denom
    w_second = p2 / denom

    e_all = jnp.concatenate([i1, i2], axis=0)
    w_all = jnp.concatenate([w_first, w_second], axis=0)

    piota = lax.broadcasted_iota(jnp.int32, (_NP, _E), 1)
    M = (e_all == piota).astype(jnp.float32)

    r0 = lax.broadcasted_iota(jnp.int32, (_NP, _NP), 0)
    r1 = lax.broadcasted_iota(jnp.int32, (_NP, _NP), 1)
    L = (r1 < r0).astype(jnp.float32)
    dn = (((1,), (0,)), ((), ()))
    exclcum = lax.dot_general(L, M, dn, preferred_element_type=jnp.float32)

    cnt = jnp.sum(M, axis=0, keepdims=True)
    padded = jnp.ceil(cnt * (1.0 / _B)) * _B
    u0 = lax.broadcasted_iota(jnp.int32, (_E, _E), 0)
    u1 = lax.broadcasted_iota(jnp.int32, (_E, _E), 1)
    U = (u0 < u1).astype(jnp.float32)
    offs = lax.dot_general(padded, U, dn, preferred_element_type=jnp.float32)

    offs_p = jnp.sum(M * offs, axis=1, keepdims=True)
    rank_p = jnp.sum(M * exclcum, axis=1, keepdims=True)
    dest = offs_p + rank_p

    cum = offs + padded
    total = jnp.sum(padded, axis=1, keepdims=True)
    la = jnp.sum((cum < total).astype(jnp.float32), axis=1, keepdims=True)

    bB = lax.broadcasted_iota(jnp.int32, (1, 32), 1).astype(jnp.float32) * _B
    be = jnp.zeros((1, 32), jnp.float32)
    for e in range(_E):
        ce = lax.slice(cum, (0, e), (1, e + 1))
        be = be + (bB >= ce).astype(jnp.float32)
    be = jnp.where(bB < total, be, la)
    nab = total * (1.0 / _B)
    vend = offs + cnt
    vend32 = jnp.zeros((1, 32), jnp.float32)
    lane = lax.broadcasted_iota(jnp.int32, (1, 32), 1)
    for e in range(_E):
        ve = lax.slice(vend, (0, e), (1, e + 1))
        vend32 = vend32 + jnp.where(lane == 16 + e, ve, 0.0)
    meta = jnp.where(lane < _NB, be, jnp.where(lane < 16, nab, vend32))
    meta_ref[...] = meta.astype(jnp.int32)
    dest_ref[...] = dest.astype(jnp.int32)
    wall_ref[...] = w_all


def _route(router_logits):
    return pl.pallas_call(
        _route_kernel,
        in_specs=[pl.BlockSpec((_T, _E), lambda: (0, 0))],
        out_specs=[
            pl.BlockSpec((1, 32), lambda: (0, 0)),
            pl.BlockSpec((_NP, 1), lambda: (0, 0)),
            pl.BlockSpec((_NP, 1), lambda: (0, 0)),
        ],
        out_shape=[
            jax.ShapeDtypeStruct((1, 32), jnp.int32),
            jax.ShapeDtypeStruct((_NP, 1), jnp.int32),
            jax.ShapeDtypeStruct((_NP, 1), jnp.float32),
        ],
    )(router_logits)


@functools.cache
def _make_sc_scatter():
    @functools.partial(
        pl.kernel,
        mesh=plsc.VectorSubcoreMesh(core_axis_name="c", subcore_axis_name="s"),
        out_type=jax.ShapeDtypeStruct((_PMAX, _H), jnp.float32),
        scratch_types=[
            pltpu.VMEM((_PPW,), jnp.int32),
            pltpu.VMEM((_PPW, _H), jnp.float32),
            pltpu.SemaphoreType.DMA,
        ],
    )
    def _sc_scatter(x_hbm, dest_hbm, xs_hbm, idx_v, rows_v, sem):
        wid = lax.axis_index("s") * _NC + lax.axis_index("c")
        base = wid * _PPW
        tok_base = lax.rem(base, _T)
        pltpu.sync_copy(dest_hbm.at[pl.ds(base, _PPW)], idx_v)
        pltpu.sync_copy(x_hbm.at[pl.ds(tok_base, _PPW)], rows_v)
        pltpu.async_copy(rows_v, xs_hbm.at[idx_v], sem).wait()

    return _sc_scatter


def _mlp_kernel(meta_ref, xs_ref, dest_ref, wall_ref,
                w1_ref, w3_ref, w2_ref, out_ref, acc):
    b = pl.program_id(0)
    i = pl.program_id(1)
    nab = meta_ref[0, 12]

    @pl.when((b == 0) & (i == 0))
    def _():
        out_ref[...] = jnp.zeros_like(out_ref)

    @pl.when(b < nab)
    def _():
        xs = xs_ref[...]
        dn = (((1,), (1,)), ((), ()))
        g = lax.dot_general(xs, w1_ref[0], dn,
                            preferred_element_type=jnp.float32)
        u = lax.dot_general(xs, w3_ref[0], dn,
                            preferred_element_type=jnp.float32)
        prod = (g * jax.nn.sigmoid(g)) * u
        y = lax.dot_general(prod, w2_ref[0], dn,
                            preferred_element_type=jnp.float32)

        @pl.when(i == 0)
        def _():
            acc[...] = y

        @pl.when(i == _NI - 1)
        def _():
            yfull = acc[...] + y
            vend = meta_ref[0, 16 + meta_ref[0, b]]
            riota = (lax.broadcasted_iota(jnp.int32, (_B, 1), 0) + b * _B)
            yfull = jnp.where(riota < vend, yfull, 0.0)
            d1 = dest_ref[:_T, :]
            d2 = dest_ref[_T:, :]
            wa1 = wall_ref[:_T, :]
            wa2 = wall_ref[_T:, :]
            q = lax.broadcasted_iota(jnp.int32, (_T, _B), 1) + b * _B
            C = ((q == d1).astype(jnp.float32) * wa1
                 + (q == d2).astype(jnp.float32) * wa2)
            dnc = (((1,), (0,)), ((), ()))
            out_ref[...] += lax.dot_general(
                C, yfull, dnc, preferred_element_type=jnp.float32)


def _mlp(meta, xs_all, dest, wall, W1, W3, W2):
    def w13_map(b, i, m):
        e = m[0, b]
        ii = jnp.where(b < m[0, 12], i, _NI - 1)
        return (e, ii, 0)

    def w2_map(b, i, m):
        e = m[0, b]
        ii = jnp.where(b < m[0, 12], i, _NI - 1)
        return (e, 0, ii)

    grid_spec = pltpu.PrefetchScalarGridSpec(
        num_scalar_prefetch=1,
        grid=(_NB, _NI),
        in_specs=[
            pl.BlockSpec((_B, _H),
                         lambda b, i, m: (jnp.minimum(b, m[0, 12] - 1), 0)),
            pl.BlockSpec((_NP, 1), lambda b, i, m: (0, 0)),
            pl.BlockSpec((_NP, 1), lambda b, i, m: (0, 0)),
            pl.BlockSpec((1, _IB, _H), w13_map),
            pl.BlockSpec((1, _IB, _H), w13_map),
            pl.BlockSpec((1, _H, _IB), w2_map),
        ],
        out_specs=pl.BlockSpec((_T, _H), lambda b, i, m: (0, 0)),
        scratch_shapes=[pltpu.VMEM((_B, _H), jnp.float32)],
    )
    return pl.pallas_call(
        _mlp_kernel,
        grid_spec=grid_spec,
        out_shape=jax.ShapeDtypeStruct((_T, _H), jnp.float32),
        compiler_params=pltpu.CompilerParams(
            dimension_semantics=("arbitrary", "arbitrary"),
        ),
    )(meta, xs_all, dest, wall, W1, W3, W2)


def kernel(hidden_states, router_logits, W1, W3, W2):
    x = hidden_states.reshape(-1, _H)
    meta, dest, wall = _route(router_logits)
    dest1 = dest.reshape(_NP)
    xs_all = _make_sc_scatter()(x, dest1)
    return _mlp(meta, xs_all, dest, wall, W1, W3, W2)
